# baseline (speedup 1.0000x reference)
import functools

import jax
import jax.numpy as jnp
from jax import lax
from jax.experimental import pallas as pl
from jax.experimental.pallas import tpu as pltpu
from jax.experimental.pallas import tpu_sc as plsc

_B = 16384
_H = 50
_V = 1000
_D = 128

_NC = 2
_NS = 16
_NW = _NC * _NS
_BPT = _B // _NW
_NB = 4
_CR = _BPT // _NB
_NCH = _H * _NB


def _transform_body(e_ref, w_ref, b_ref, t_ref):
    prod = lax.dot_general(
        e_ref[...], w_ref[...], (((1,), (1,)), ((), ())),
        preferred_element_type=jnp.float32,
        precision=lax.Precision.HIGHEST)
    t_ref[...] = jnp.maximum(prod + b_ref[...], 0.0)


def _transform_table(emb_table, W, b):
    return pl.pallas_call(
        _transform_body,
        out_shape=jax.ShapeDtypeStruct((_V, _D), jnp.float32),
    )(emb_table, W, b.reshape(1, _D))


def _gather_body(table_hbm, idx_hbm, out_hbm, tbl_sh, idx_v,
                 stga, stgb, stgc, stgd, isem,
                 ga, gb, gc, gd, oa, ob, oc, od):
    wid = lax.axis_index("s") * _NC + lax.axis_index("c")
    b0 = wid * _BPT
    @pl.when(lax.axis_index("s") == 0)
    def _():
        pltpu.sync_copy(table_hbm, tbl_sh)

    def idx_dma(h):
        return pltpu.make_async_copy(
            idx_hbm.at[h, pl.ds(b0, _BPT)],
            idx_v.at[pl.ds(h * _BPT, _BPT)], isem)

    def fire_idx(h, carry):
        idx_dma(h).start()
        return carry

    def drain_idx(h, carry):
        idx_dma(h).wait()
        return carry

    lax.fori_loop(0, _H, fire_idx, 0)
    lax.fori_loop(0, _H, drain_idx, 0)
    plsc.subcore_barrier()

    stgs = (stga, stgb, stgc, stgd)
    gsems = (ga, gb, gc, gd)
    osems = (oa, ob, oc, od)

    def gather(c, k):
        return pltpu.make_async_copy(
            tbl_sh.at[idx_v.at[pl.ds(c * _CR, _CR)]], stgs[k], gsems[k])

    def out_copy(c, k):
        h = lax.shift_right_logical(c, 2)
        q = lax.bitwise_and(c, 3)
        return pltpu.make_async_copy(
            stgs[k], out_hbm.at[h, pl.ds(b0 + q * _CR, _CR), :], osems[k])

    for k in range(_NB):
        gather(k, k).start()

    niter = _NCH // _NB

    def body(i, carry):
        c = _NB * i
        for k in range(_NB):
            gather(c + k, k).wait()
            out_copy(c + k, k).start()
        for k in range(_NB):
            out_copy(c + k, k).wait()

        @pl.when(i < niter - 1)
        def _():
            for k in range(_NB):
                gather(c + k + _NB, k).start()

        return carry

    lax.fori_loop(0, niter, body, 0)


def _gather(table, idx_t):
    mesh = plsc.VectorSubcoreMesh(core_axis_name="c", subcore_axis_name="s")
    run = functools.partial(
        pl.kernel,
        mesh=mesh,
        compiler_params=pltpu.CompilerParams(needs_layout_passes=False),
        out_type=jax.ShapeDtypeStruct((_H, _B, _D), jnp.float32),
        scratch_types=[
            pltpu.VMEM_SHARED((_V, _D), jnp.float32),
            pltpu.VMEM((_H * _BPT,), jnp.int32),
            pltpu.VMEM((_CR, _D), jnp.float32),
            pltpu.VMEM((_CR, _D), jnp.float32),
            pltpu.VMEM((_CR, _D), jnp.float32),
            pltpu.VMEM((_CR, _D), jnp.float32),
            pltpu.SemaphoreType.DMA,
            pltpu.SemaphoreType.DMA,
            pltpu.SemaphoreType.DMA,
            pltpu.SemaphoreType.DMA,
            pltpu.SemaphoreType.DMA,
            pltpu.SemaphoreType.DMA,
            pltpu.SemaphoreType.DMA,
            pltpu.SemaphoreType.DMA,
            pltpu.SemaphoreType.DMA,
        ],
    )(_gather_body)
    return run(table, idx_t)


def kernel(industry_ids, emb_table, W, b):
    table = _transform_table(emb_table, W, b)
    idx_t = industry_ids.astype(jnp.int32).T
    out_t = _gather(table, idx_t)
    return jnp.transpose(out_t, (1, 0, 2))


# speedup vs baseline: 1.2057x; 1.2057x over previous
import functools

import jax
import jax.numpy as jnp
from jax import lax
from jax.experimental import pallas as pl
from jax.experimental.pallas import tpu as pltpu
from jax.experimental.pallas import tpu_sc as plsc

_B = 16384
_H = 50
_V = 1000
_D = 128

_NC = 2
_NS = 16
_NW = _NC * _NS
_BPT = _B // _NW
_CR = 256
_NCH = _H * 2


def _transform_body(e_ref, w_ref, b_ref, t_ref):
    prod = lax.dot_general(
        e_ref[...], w_ref[...], (((1,), (1,)), ((), ())),
        preferred_element_type=jnp.float32,
        precision=lax.Precision.HIGHEST)
    t_ref[...] = jnp.maximum(prod + b_ref[...], 0.0)


def _transform_table(emb_table, W, b):
    return pl.pallas_call(
        _transform_body,
        out_shape=jax.ShapeDtypeStruct((_V, _D), jnp.float32),
    )(emb_table, W, b.reshape(1, _D))


def _gather_body(table_hbm, idx_hbm, out_hbm, tbl_sh, idx_v, stga, stgb,
                 isem, ga, gb, oa, ob):
    wid = lax.axis_index("s") * _NC + lax.axis_index("c")
    b0 = wid * _BPT
    @pl.when(lax.axis_index("s") == 0)
    def _():
        pltpu.sync_copy(table_hbm, tbl_sh)

    def idx_dma(h):
        return pltpu.make_async_copy(
            idx_hbm.at[h, pl.ds(b0, _BPT)],
            idx_v.at[pl.ds(h * _BPT, _BPT)], isem)

    def fire_idx(h, carry):
        idx_dma(h).start()
        return carry

    def drain_idx(h, carry):
        idx_dma(h).wait()
        return carry

    lax.fori_loop(0, _H, fire_idx, 0)
    lax.fori_loop(0, _H, drain_idx, 0)
    plsc.subcore_barrier()

    def gather(h, half, stg, sem):
        h = jnp.minimum(h, _H - 1)
        return pltpu.make_async_copy(
            tbl_sh.at[idx_v.at[pl.ds(h * (2 * _CR) + half * _CR, _CR)]],
            stg, sem)

    def out_copy(h, half, stg, sem):
        return pltpu.make_async_copy(
            stg, out_hbm.at[h, pl.ds(b0 + half * _CR, _CR), :], sem)

    gather(0, 0, stga, ga).start()

    def body(h, carry):
        gather(h, 0, stga, ga).wait()

        @pl.when(h > 0)
        def _():
            out_copy(h - 1, 1, stgb, ob).wait()

        gather(h, 1, stgb, gb).start()
        out_copy(h, 0, stga, oa).start()
        gather(h, 1, stgb, gb).wait()
        out_copy(h, 0, stga, oa).wait()
        gather(h + 1, 0, stga, ga).start()
        out_copy(h, 1, stgb, ob).start()
        return carry

    lax.fori_loop(0, _H, body, 0)
    out_copy(_H - 1, 1, stgb, ob).wait()
    gather(_H, 0, stga, ga).wait()


def _gather(table, idx_t):
    mesh = plsc.VectorSubcoreMesh(core_axis_name="c", subcore_axis_name="s")
    run = functools.partial(
        pl.kernel,
        mesh=mesh,
        compiler_params=pltpu.CompilerParams(needs_layout_passes=False),
        out_type=jax.ShapeDtypeStruct((_H, _B, _D), jnp.float32),
        scratch_types=[
            pltpu.VMEM_SHARED((_V, _D), jnp.float32),
            pltpu.VMEM((_H * _BPT,), jnp.int32),
            pltpu.VMEM((_CR, _D), jnp.float32),
            pltpu.VMEM((_CR, _D), jnp.float32),
            pltpu.SemaphoreType.DMA,
            pltpu.SemaphoreType.DMA,
            pltpu.SemaphoreType.DMA,
            pltpu.SemaphoreType.DMA,
            pltpu.SemaphoreType.DMA,
        ],
    )(_gather_body)
    return run(table, idx_t)


def kernel(industry_ids, emb_table, W, b):
    table = _transform_table(emb_table, W, b)
    idx_t = industry_ids.astype(jnp.int32).T
    out_t = _gather(table, idx_t)
    return jnp.transpose(out_t, (1, 0, 2))
